# baseline (speedup 1.0000x reference)
import functools

import jax
import jax.numpy as jnp
from jax import lax
from jax.experimental import pallas as pl
from jax.experimental.pallas import tpu as pltpu
from jax.experimental.pallas import tpu_sc as plsc

EMBED_DIM = 32
NUM_CORES = 2
NUM_SUBCORES = 16
NW = NUM_CORES * NUM_SUBCORES
IDX_LANE = 128
CHUNK = 512
VPC = CHUNK // IDX_LANE
L = 16


@functools.lru_cache(maxsize=None)
def _make_gather(n_b1: int, n_b2: int):
    n_idx = n_b1 * n_b2
    n_vecs = n_idx // IDX_LANE
    vecs_per_w = n_vecs // NW
    chunks_per_w = n_idx // (NW * CHUNK)
    chunks_per_b2 = n_b1 // CHUNK

    mesh = plsc.VectorSubcoreMesh(
        core_axis_name="c", subcore_axis_name="s",
        num_cores=NUM_CORES, num_subcores=NUM_SUBCORES)

    @functools.partial(
        pl.kernel,
        mesh=mesh,
        out_type=jax.ShapeDtypeStruct((n_b2, EMBED_DIM, n_b1), jnp.float32),
        scratch_types=[
            pltpu.VMEM((vecs_per_w, IDX_LANE), jnp.int32),
            pltpu.VMEM((2, CHUNK, EMBED_DIM), jnp.float32),
            pltpu.VMEM((2, EMBED_DIM, CHUNK), jnp.float32),
            pltpu.SemaphoreType.DMA,
            pltpu.SemaphoreType.DMA,
        ],
        compiler_params=pltpu.CompilerParams(
            use_tc_tiling_on_sc=False, needs_layout_passes=False),
    )
    def gather_kernel(idx_hbm, table_hbm, out_hbm, idx_v, rows_v, tbuf,
                      gsem, osem):
        wid = lax.axis_index("s") * NUM_CORES + lax.axis_index("c")
        g0 = wid * chunks_per_w
        pltpu.sync_copy(idx_hbm.at[pl.ds(wid * vecs_per_w, vecs_per_w)],
                        idx_v)
        iota = lax.iota(jnp.int32, L)

        def fire(c, slot):
            for k in range(VPC):
                pltpu.async_copy(
                    table_hbm.at[idx_v.at[c * VPC + k]],
                    rows_v.at[slot].at[pl.ds(k * IDX_LANE, IDX_LANE)],
                    gsem)

        def drain(slot):
            for k in range(VPC):
                pltpu.make_async_copy(
                    table_hbm.at[idx_v.at[k]],
                    rows_v.at[slot].at[pl.ds(k * IDX_LANE, IDX_LANE)],
                    gsem).wait()

        def out_dst(g):
            b2 = g // chunks_per_b2
            b1o = (g % chunks_per_b2) * CHUNK
            return out_hbm.at[b2, :, pl.ds(b1o, CHUNK)]

        def owait():
            pltpu.make_async_copy(tbuf.at[0], out_dst(0), osem).wait()

        fire(0, 0)

        @pl.loop(0, chunks_per_w)
        def _chunk(c):
            slot = lax.rem(c, 2)
            drain(slot)

            @pl.when(c + 1 < chunks_per_w)
            def _fire_next():
                fire(c + 1, lax.rem(c + 1, 2))

            @pl.when(c >= 2)
            def _wait_out():
                owait()

            @pl.loop(0, EMBED_DIM)
            def _transpose(d):
                col = jnp.full((L,), d, jnp.int32)
                for c16 in range(CHUNK // L):
                    rid = iota + (c16 * L)
                    tbuf.at[slot, d][pl.ds(c16 * L, L)] = plsc.load_gather(
                        rows_v.at[slot], [rid, col])

            pltpu.async_copy(tbuf.at[slot], out_dst(g0 + c), osem)

        owait()
        owait()

    return gather_kernel


def kernel(x, table):
    n_b1, n_b2 = x.shape
    idx2d = x.T.astype(jnp.int32).reshape(-1, IDX_LANE)
    out = _make_gather(n_b1, n_b2)(idx2d, table)
    return jnp.transpose(out, (2, 0, 1))


# speedup vs baseline: 1.1207x; 1.1207x over previous
import functools

import jax
import jax.numpy as jnp
from jax import lax
from jax.experimental import pallas as pl
from jax.experimental.pallas import tpu as pltpu
from jax.experimental.pallas import tpu_sc as plsc

EMBED_DIM = 32
NUM_CORES = 2
NUM_SUBCORES = 16
NW = NUM_CORES * NUM_SUBCORES
B_B1 = 64


@functools.lru_cache(maxsize=None)
def _make_gather(n_b1: int, n_b2: int):
    rows_per_w = n_b1 // NW
    n_blocks = rows_per_w // B_B1

    mesh = plsc.VectorSubcoreMesh(
        core_axis_name="c", subcore_axis_name="s",
        num_cores=NUM_CORES, num_subcores=NUM_SUBCORES)

    @functools.partial(
        pl.kernel,
        mesh=mesh,
        out_type=jax.ShapeDtypeStruct((n_b1, n_b2, EMBED_DIM), jnp.float32),
        scratch_types=[
            pltpu.VMEM((rows_per_w, n_b2), jnp.int32),
            pltpu.VMEM((2, B_B1, n_b2, EMBED_DIM), jnp.float32),
            pltpu.SemaphoreType.DMA,
            pltpu.SemaphoreType.DMA,
        ],
        compiler_params=pltpu.CompilerParams(use_tc_tiling_on_sc=False),
    )
    def gather_kernel(idx_hbm, table_hbm, out_hbm, idx_v, rows_v, gsem, osem):
        wid = lax.axis_index("s") * NUM_CORES + lax.axis_index("c")
        pltpu.sync_copy(idx_hbm.at[pl.ds(wid * rows_per_w, rows_per_w)],
                        idx_v)

        def fire(b, slot):
            @pl.loop(0, B_B1)
            def _fire_row(r):
                pltpu.async_copy(
                    table_hbm.at[idx_v.at[b * B_B1 + r]],
                    rows_v.at[slot, r], gsem)

        def drain(slot):
            @pl.loop(0, B_B1)
            def _drain_row(r):
                pltpu.make_async_copy(
                    table_hbm.at[idx_v.at[0]],
                    rows_v.at[slot, 0], gsem).wait()

        def out_dst(b):
            return out_hbm.at[pl.ds(wid * rows_per_w + b * B_B1, B_B1)]

        def owait():
            pltpu.make_async_copy(rows_v.at[0], out_dst(0), osem).wait()

        fire(0, 0)

        @pl.loop(0, n_blocks)
        def _block(b):
            slot = lax.rem(b, 2)
            drain(slot)

            @pl.when(b + 1 < n_blocks)
            def _fire_next():
                fire(b + 1, lax.rem(b + 1, 2))

            @pl.when(b >= 2)
            def _wait_out():
                owait()

            pltpu.async_copy(rows_v.at[slot], out_dst(b), osem)

        owait()
        owait()

    return gather_kernel


def kernel(x, table):
    n_b1, n_b2 = x.shape
    return _make_gather(n_b1, n_b2)(x.astype(jnp.int32), table)
